# Initial kernel scaffold; baseline (speedup 1.0000x reference)
import jax
import jax.numpy as jnp
from jax.experimental import pallas as pl


def kernel(x, edge_index, W1, att_src1, att_dst1, b1, W2, att_src2, att_dst2, b2):
    raise NotImplementedError("write your pallas kernel here")


# speedup vs baseline: 40.4554x; 40.4554x over previous
import functools

import jax
import jax.numpy as jnp
from jax import lax
from jax.experimental import pallas as pl
from jax.experimental.pallas import tpu as pltpu
from jax.experimental.pallas import tpu_sc as plsc

N = 10000
F_IN = 128
H1 = 8
C_HID = 8
D1 = H1 * C_HID
C_OUT = 40
D2P = 48
E = 320000
E2 = E + N
NC, NS, LANES = 2, 16, 16
NW = NC * NS
NP = 10240
CHUNK = 512
CQ = CHUNK // 128
TE = 21 * CHUNK
EP = NW * TE
NPT = NP // NS

f32 = jnp.float32

_MESH = plsc.VectorSubcoreMesh(
    core_axis_name="c", subcore_axis_name="s", num_cores=NC, num_subcores=NS
)

_GATHER_DNUMS = lax.GatherDimensionNumbers(
    offset_dims=(), collapsed_slice_dims=(0,), start_index_map=(0,)
)


def _perm16(v, idx):
    return lax.gather(
        v, idx.reshape(16, 1), _GATHER_DNUMS, (1,),
        mode=lax.GatherScatterMode.PROMISE_IN_BOUNDS,
    )


def _l1_body(src_hbm, dst_hbm, acat_hbm, h_hbm, z64_hbm, z16_hbm,
             out_hbm, den_hbm,
             src_v, dst_v, srow_v, drow_v, hrow_v,
             oacc, dacc, sem_s, sem_d, sem_h, sem_w):
    cid = lax.axis_index("c")
    sid = lax.axis_index("s")
    wid = sid * NC + cid
    row0 = sid * NPT

    pltpu.sync_copy(z64_hbm.at[pl.ds(row0, NPT)], oacc.at[pl.ds(row0, NPT)])
    pltpu.sync_copy(z16_hbm.at[pl.ds(row0, NPT)], dacc.at[pl.ds(row0, NPT)])
    plsc.subcore_barrier()

    lane = lax.iota(jnp.int32, 16)
    rot8 = 8 + (lane & 7)
    half = lane >> 3
    scale_idx = [2 * q + half for q in range(4)]

    def chunk(k, carry):
        rbase = wid * (TE // 128) + k * CQ
        base = wid * TE + k * CHUNK
        pltpu.sync_copy(src_hbm.at[pl.ds(rbase, CQ)], src_v)
        pltpu.sync_copy(dst_hbm.at[pl.ds(rbase, CQ)], dst_v)
        cps = []
        for j in range(CQ):
            rows = pl.ds(j * 128, 128)
            cps.append(pltpu.async_copy(
                acat_hbm.at[src_v.at[j]], srow_v.at[rows], sem_s))
            cps.append(pltpu.async_copy(
                acat_hbm.at[dst_v.at[j]], drow_v.at[rows], sem_d))
            cps.append(pltpu.async_copy(
                h_hbm.at[src_v.at[j]], hrow_v.at[rows], sem_h))
        for cp in cps:
            cp.wait()

        def edge(e, c2):
            sr = srow_v[e, :]
            dr = drow_v[e, :]
            al = sr + _perm16(dr, rot8)
            al = jnp.maximum(al, 0.2 * al)
            ex = jnp.exp(al)
            wt = ((base + e) < E2).astype(f32)
            ex = ex * wt
            srow_v[e, :] = ex
            for q in range(4):
                scq = _perm16(ex, scale_idx[q])
                cols = pl.ds(16 * q, 16)
                hrow_v[e, cols] = hrow_v[e, cols] * scq
            return c2

        lax.fori_loop(0, CHUNK, edge, 0)

        wps = []
        for j in range(CQ):
            rows = pl.ds(j * 128, 128)
            wps.append(pltpu.async_copy(
                srow_v.at[rows], dacc.at[dst_v.at[j]], sem_w, add=True))
            wps.append(pltpu.async_copy(
                hrow_v.at[rows], oacc.at[dst_v.at[j]], sem_w, add=True))
        for wp in wps:
            wp.wait()
        return carry

    lax.fori_loop(0, TE // CHUNK, chunk, 0)
    plsc.subcore_barrier()
    pltpu.sync_copy(oacc.at[pl.ds(row0, NPT)],
                    out_hbm.at[cid, pl.ds(row0, NPT)])
    pltpu.sync_copy(dacc.at[pl.ds(row0, NPT)],
                    den_hbm.at[cid, pl.ds(row0, NPT)])


_l1_edge = functools.partial(
    pl.kernel,
    out_type=(
        jax.ShapeDtypeStruct((NC, NP, D1), f32),
        jax.ShapeDtypeStruct((NC, NP, 16), f32),
    ),
    mesh=_MESH,
    compiler_params=pltpu.CompilerParams(use_tc_tiling_on_sc=False, needs_layout_passes=False),
    scratch_types=[
        pltpu.VMEM((CQ, 128), jnp.int32),
        pltpu.VMEM((CQ, 128), jnp.int32),
        pltpu.VMEM((CHUNK, 16), f32),
        pltpu.VMEM((CHUNK, 16), f32),
        pltpu.VMEM((CHUNK, D1), f32),
        pltpu.VMEM_SHARED((NP, D1), f32),
        pltpu.VMEM_SHARED((NP, 16), f32),
        pltpu.SemaphoreType.DMA,
        pltpu.SemaphoreType.DMA,
        pltpu.SemaphoreType.DMA,
        pltpu.SemaphoreType.DMA,
    ],
)(_l1_body)


def _l2_body(src_hbm, dst_hbm, asrc_hbm, adst_hbm, h_hbm, z48_hbm, z16_hbm,
             out_hbm, den_hbm,
             src_v, dst_v, asrc_v, adst_v, hrow_v, exrow_v,
             oacc, dacc, sem_h, sem_w):
    cid = lax.axis_index("c")
    sid = lax.axis_index("s")
    wid = sid * NC + cid
    row0 = sid * NPT

    pltpu.sync_copy(asrc_hbm, asrc_v)
    pltpu.sync_copy(adst_hbm, adst_v)
    pltpu.sync_copy(z48_hbm.at[pl.ds(row0, NPT)], oacc.at[pl.ds(row0, NPT)])
    pltpu.sync_copy(z16_hbm.at[pl.ds(row0, NPT)], dacc.at[pl.ds(row0, NPT)])
    plsc.subcore_barrier()

    def chunk(k, carry):
        rbase = wid * (TE // 128) + k * CQ
        base = wid * TE + k * CHUNK
        pltpu.sync_copy(src_hbm.at[pl.ds(rbase, CQ)], src_v)
        pltpu.sync_copy(dst_hbm.at[pl.ds(rbase, CQ)], dst_v)
        cps = []
        for j in range(CQ):
            rows = pl.ds(j * 128, 128)
            cps.append(pltpu.async_copy(
                h_hbm.at[src_v.at[j]], hrow_v.at[rows], sem_h))
        for cp in cps:
            cp.wait()

        def grp(g, c2):
            src16 = src_v[g // 8, pl.ds((g % 8) * 16, 16)]
            dst16 = dst_v[g // 8, pl.ds((g % 8) * 16, 16)]
            a_s = plsc.load_gather(asrc_v, [src16])
            a_d = plsc.load_gather(adst_v, [dst16])
            al = a_s + a_d
            al = jnp.maximum(al, 0.2 * al)
            ex = jnp.exp(al)
            gid = base + g * 16 + lax.iota(jnp.int32, 16)
            ex = ex * (gid < E2).astype(f32)
            for l in range(16):
                e = g * 16 + l
                sc = _perm16(ex, jnp.full((16,), l, jnp.int32))
                exrow_v[e, :] = sc
                for q in range(3):
                    cols = pl.ds(16 * q, 16)
                    hrow_v[e, cols] = hrow_v[e, cols] * sc
            return c2

        lax.fori_loop(0, CHUNK // 16, grp, 0)

        wps = []
        for j in range(CQ):
            rows = pl.ds(j * 128, 128)
            wps.append(pltpu.async_copy(
                exrow_v.at[rows], dacc.at[dst_v.at[j]], sem_w, add=True))
            wps.append(pltpu.async_copy(
                hrow_v.at[rows], oacc.at[dst_v.at[j]], sem_w, add=True))
        for wp in wps:
            wp.wait()
        return carry

    lax.fori_loop(0, TE // CHUNK, chunk, 0)
    plsc.subcore_barrier()
    pltpu.sync_copy(oacc.at[pl.ds(row0, NPT)],
                    out_hbm.at[cid, pl.ds(row0, NPT)])
    pltpu.sync_copy(dacc.at[pl.ds(row0, NPT)],
                    den_hbm.at[cid, pl.ds(row0, NPT)])


_l2_edge = functools.partial(
    pl.kernel,
    out_type=(
        jax.ShapeDtypeStruct((NC, NP, D2P), f32),
        jax.ShapeDtypeStruct((NC, NP, 16), f32),
    ),
    mesh=_MESH,
    compiler_params=pltpu.CompilerParams(use_tc_tiling_on_sc=False, needs_layout_passes=False),
    scratch_types=[
        pltpu.VMEM((CQ, 128), jnp.int32),
        pltpu.VMEM((CQ, 128), jnp.int32),
        pltpu.VMEM((NP,), f32),
        pltpu.VMEM((NP,), f32),
        pltpu.VMEM((CHUNK, D2P), f32),
        pltpu.VMEM((CHUNK, 16), f32),
        pltpu.VMEM_SHARED((NP, D2P), f32),
        pltpu.VMEM_SHARED((NP, 16), f32),
        pltpu.SemaphoreType.DMA,
        pltpu.SemaphoreType.DMA,
    ],
)(_l2_body)


def _dense1_body(x_ref, w1_ref, ab1_ref, h1_ref, acat_ref):
    h1 = jnp.dot(x_ref[...], w1_ref[...], preferred_element_type=f32)
    h1_ref[...] = h1
    acat_ref[...] = jnp.dot(h1, ab1_ref[...], preferred_element_type=f32)


def _dense2_body(op_ref, dp_ref, b1_ref, w2_ref, a2_ref, r16_ref,
                 h2_ref, acat2_ref):
    osum = op_ref[0] + op_ref[1]
    dsum = dp_ref[0] + dp_ref[1]
    den = jnp.dot(dsum, r16_ref[...], preferred_element_type=f32)
    out1 = osum / (den + 1e-16) + b1_ref[...]
    r = jnp.maximum(out1, 0.0)
    h2 = jnp.dot(r, w2_ref[...], preferred_element_type=f32)
    h2_ref[...] = h2
    acat2_ref[...] = jnp.dot(h2, a2_ref[...], preferred_element_type=f32)


def _dense3_body(op_ref, dp_ref, b2_ref, out_ref):
    osum = op_ref[0] + op_ref[1]
    den = dp_ref[0][:, 0:1] + dp_ref[1][:, 0:1]
    logits = osum[:, 0:C_OUT] / (den + 1e-16) + b2_ref[...]
    m = jnp.max(logits, axis=1, keepdims=True)
    s = logits - m
    lse = jnp.log(jnp.sum(jnp.exp(s), axis=1, keepdims=True))
    out_ref[...] = s - lse


def kernel(x, edge_index, W1, att_src1, att_dst1, b1,
           W2, att_src2, att_dst2, b2):
    loop = jnp.arange(N, dtype=jnp.int32)
    pad = jnp.zeros((EP - E2,), dtype=jnp.int32)
    src = jnp.concatenate([edge_index[0].astype(jnp.int32), loop, pad])
    dst = jnp.concatenate([edge_index[1].astype(jnp.int32), loop, pad])
    src2 = src.reshape(EP // 128, 128)
    dst2 = dst.reshape(EP // 128, 128)

    eye8 = jnp.eye(H1, dtype=f32)
    a_src_m = jnp.einsum("hc,hk->hck", att_src1, eye8).reshape(D1, H1)
    a_dst_m = jnp.einsum("hc,hk->hck", att_dst1, eye8).reshape(D1, H1)
    AB1 = jnp.concatenate([a_src_m, a_dst_m], axis=1)
    R16 = jnp.concatenate(
        [jnp.kron(eye8, jnp.ones((1, C_HID), f32)),
         jnp.zeros((8, D1), f32)], axis=0)
    W2p = jnp.concatenate(
        [W2, jnp.zeros((D1, D2P - C_OUT), f32)], axis=1)
    A2 = jnp.zeros((D2P, 16), f32)
    A2 = A2.at[:C_OUT, 0].set(att_src2[0]).at[:C_OUT, 1].set(att_dst2[0])
    b1r = b1.reshape(1, D1)
    b2r = b2.reshape(1, C_OUT)
    z64 = jnp.zeros((NP, D1), f32)
    z48 = jnp.zeros((NP, D2P), f32)
    z16 = jnp.zeros((NP, 16), f32)

    h1, acat1 = pl.pallas_call(
        _dense1_body,
        out_shape=(jax.ShapeDtypeStruct((N, D1), f32),
                   jax.ShapeDtypeStruct((N, 16), f32)),
    )(x, W1, AB1)

    op1, dp1 = _l1_edge(src2, dst2, acat1, h1, z64, z16)

    h2p, acat2 = pl.pallas_call(
        _dense2_body,
        out_shape=(jax.ShapeDtypeStruct((NP, D2P), f32),
                   jax.ShapeDtypeStruct((NP, 16), f32)),
    )(op1, dp1, b1r, W2p, A2, R16)

    asrc2 = acat2[:, 0]
    adst2 = acat2[:, 1]

    op2, dp2 = _l2_edge(src2, dst2, asrc2, adst2, h2p, z48, z16)

    out = pl.pallas_call(
        _dense3_body,
        out_shape=jax.ShapeDtypeStruct((NP, C_OUT), f32),
    )(op2, dp2, b2r)
    return out[:N]


# speedup vs baseline: 59.9417x; 1.4817x over previous
import functools

import jax
import jax.numpy as jnp
from jax import lax
from jax.experimental import pallas as pl
from jax.experimental.pallas import tpu as pltpu
from jax.experimental.pallas import tpu_sc as plsc

N = 10000
F_IN = 128
H1 = 8
C_HID = 8
D1 = H1 * C_HID
C_OUT = 40
D2P = 48
E = 320000
E2 = E + N
NC, NS, LANES = 2, 16, 16
NW = NC * NS
NP = 10240
CHUNK = 512
CQ = CHUNK // 128
TE = 21 * CHUNK
EP = NW * TE
NPT = NP // NS

f32 = jnp.float32

_MESH = plsc.VectorSubcoreMesh(
    core_axis_name="c", subcore_axis_name="s", num_cores=NC, num_subcores=NS
)

_GATHER_DNUMS = lax.GatherDimensionNumbers(
    offset_dims=(), collapsed_slice_dims=(0,), start_index_map=(0,)
)


def _perm16(v, idx):
    return lax.gather(
        v, idx.reshape(16, 1), _GATHER_DNUMS, (1,),
        mode=lax.GatherScatterMode.PROMISE_IN_BOUNDS,
    )


def _l1_body(src_hbm, dst_hbm, acats_hbm, acatd_hbm, h_hbm, z64_hbm, z16_hbm,
             out_hbm, den_hbm,
             src_v, dst_v, srow_v, drow_v, hrow_v,
             oacc, dacc, sem_s, sem_d, sem_h, sem_w):
    cid = lax.axis_index("c")
    sid = lax.axis_index("s")
    wid = sid * NC + cid
    row0 = sid * NPT

    pltpu.sync_copy(z64_hbm.at[pl.ds(row0, NPT)], oacc.at[pl.ds(row0, NPT)])
    pltpu.sync_copy(z16_hbm.at[pl.ds(row0, NPT)], dacc.at[pl.ds(row0, NPT)])
    plsc.subcore_barrier()

    lane = lax.iota(jnp.int32, 16)
    half = lane >> 3
    scale_idx = [2 * q + half for q in range(4)]

    def chunk(k, carry):
        rbase = wid * (TE // 128) + k * CQ
        pltpu.sync_copy(src_hbm.at[pl.ds(rbase, CQ)], src_v)
        pltpu.sync_copy(dst_hbm.at[pl.ds(rbase, CQ)], dst_v)
        cps = []
        for j in range(CQ):
            rows = pl.ds(j * 128, 128)
            cps.append(pltpu.async_copy(
                acats_hbm.at[src_v.at[j]], srow_v.at[rows], sem_s))
            cps.append(pltpu.async_copy(
                acatd_hbm.at[dst_v.at[j]], drow_v.at[rows], sem_d))
            cps.append(pltpu.async_copy(
                h_hbm.at[src_v.at[j]], hrow_v.at[rows], sem_h))
        for cp in cps:
            cp.wait()

        @plsc.parallel_loop(0, CHUNK, unroll=8)
        def edge(e):
            al = srow_v[e, :] + drow_v[e, :]
            al = jnp.maximum(al, 0.2 * al)
            ex = jnp.exp(al)
            srow_v[e, :] = ex
            for q in range(4):
                scq = _perm16(ex, scale_idx[q])
                cols = pl.ds(16 * q, 16)
                hrow_v[e, cols] = hrow_v[e, cols] * scq

        wps = []
        for j in range(CQ):
            rows = pl.ds(j * 128, 128)
            wps.append(pltpu.async_copy(
                srow_v.at[rows], dacc.at[dst_v.at[j]], sem_w, add=True))
            wps.append(pltpu.async_copy(
                hrow_v.at[rows], oacc.at[dst_v.at[j]], sem_w, add=True))
        for wp in wps:
            wp.wait()
        return carry

    lax.fori_loop(0, TE // CHUNK, chunk, 0)
    plsc.subcore_barrier()
    pltpu.sync_copy(oacc.at[pl.ds(row0, NPT)],
                    out_hbm.at[cid, pl.ds(row0, NPT)])
    pltpu.sync_copy(dacc.at[pl.ds(row0, NPT)],
                    den_hbm.at[cid, pl.ds(row0, NPT)])


_l1_edge = functools.partial(
    pl.kernel,
    out_type=(
        jax.ShapeDtypeStruct((NC, NP, D1), f32),
        jax.ShapeDtypeStruct((NC, NP, 16), f32),
    ),
    mesh=_MESH,
    compiler_params=pltpu.CompilerParams(use_tc_tiling_on_sc=False, needs_layout_passes=False),
    scratch_types=[
        pltpu.VMEM((CQ, 128), jnp.int32),
        pltpu.VMEM((CQ, 128), jnp.int32),
        pltpu.VMEM((CHUNK, 16), f32),
        pltpu.VMEM((CHUNK, 16), f32),
        pltpu.VMEM((CHUNK, D1), f32),
        pltpu.VMEM_SHARED((NP, D1), f32),
        pltpu.VMEM_SHARED((NP, 16), f32),
        pltpu.SemaphoreType.DMA,
        pltpu.SemaphoreType.DMA,
        pltpu.SemaphoreType.DMA,
        pltpu.SemaphoreType.DMA,
    ],
)(_l1_body)


def _l2_body(src_hbm, dst_hbm, asrc_hbm, adst_hbm, h_hbm, z48_hbm, z16_hbm,
             out_hbm, den_hbm,
             src_v, dst_v, asrc_v, adst_v, hrow_v, exrow_v,
             oacc, dacc, sem_h, sem_w):
    cid = lax.axis_index("c")
    sid = lax.axis_index("s")
    wid = sid * NC + cid
    row0 = sid * NPT

    pltpu.sync_copy(asrc_hbm, asrc_v)
    pltpu.sync_copy(adst_hbm, adst_v)
    pltpu.sync_copy(z48_hbm.at[pl.ds(row0, NPT)], oacc.at[pl.ds(row0, NPT)])
    pltpu.sync_copy(z16_hbm.at[pl.ds(row0, NPT)], dacc.at[pl.ds(row0, NPT)])
    plsc.subcore_barrier()

    def chunk(k, carry):
        rbase = wid * (TE // 128) + k * CQ
        pltpu.sync_copy(src_hbm.at[pl.ds(rbase, CQ)], src_v)
        pltpu.sync_copy(dst_hbm.at[pl.ds(rbase, CQ)], dst_v)
        cps = []
        for j in range(CQ):
            rows = pl.ds(j * 128, 128)
            cps.append(pltpu.async_copy(
                h_hbm.at[src_v.at[j]], hrow_v.at[rows], sem_h))
        for cp in cps:
            cp.wait()

        @plsc.parallel_loop(0, CHUNK // 16, unroll=2)
        def grp(g):
            src16 = src_v[g // 8, pl.ds((g % 8) * 16, 16)]
            dst16 = dst_v[g // 8, pl.ds((g % 8) * 16, 16)]
            a_s = plsc.load_gather(asrc_v, [src16])
            a_d = plsc.load_gather(adst_v, [dst16])
            al = a_s + a_d
            al = jnp.maximum(al, 0.2 * al)
            ex = jnp.exp(al)
            for l in range(16):
                e = g * 16 + l
                sc = _perm16(ex, jnp.full((16,), l, jnp.int32))
                exrow_v[e, :] = sc
                for q in range(3):
                    cols = pl.ds(16 * q, 16)
                    hrow_v[e, cols] = hrow_v[e, cols] * sc

        wps = []
        for j in range(CQ):
            rows = pl.ds(j * 128, 128)
            wps.append(pltpu.async_copy(
                exrow_v.at[rows], dacc.at[dst_v.at[j]], sem_w, add=True))
            wps.append(pltpu.async_copy(
                hrow_v.at[rows], oacc.at[dst_v.at[j]], sem_w, add=True))
        for wp in wps:
            wp.wait()
        return carry

    lax.fori_loop(0, TE // CHUNK, chunk, 0)
    plsc.subcore_barrier()
    pltpu.sync_copy(oacc.at[pl.ds(row0, NPT)],
                    out_hbm.at[cid, pl.ds(row0, NPT)])
    pltpu.sync_copy(dacc.at[pl.ds(row0, NPT)],
                    den_hbm.at[cid, pl.ds(row0, NPT)])


_l2_edge = functools.partial(
    pl.kernel,
    out_type=(
        jax.ShapeDtypeStruct((NC, NP, D2P), f32),
        jax.ShapeDtypeStruct((NC, NP, 16), f32),
    ),
    mesh=_MESH,
    compiler_params=pltpu.CompilerParams(use_tc_tiling_on_sc=False, needs_layout_passes=False),
    scratch_types=[
        pltpu.VMEM((CQ, 128), jnp.int32),
        pltpu.VMEM((CQ, 128), jnp.int32),
        pltpu.VMEM((NP,), f32),
        pltpu.VMEM((NP,), f32),
        pltpu.VMEM((CHUNK, D2P), f32),
        pltpu.VMEM((CHUNK, 16), f32),
        pltpu.VMEM_SHARED((NP, D2P), f32),
        pltpu.VMEM_SHARED((NP, 16), f32),
        pltpu.SemaphoreType.DMA,
        pltpu.SemaphoreType.DMA,
    ],
)(_l2_body)


def _dense1_body(x_ref, w1_ref, as_ref, ad_ref, h1_ref, acats_ref, acatd_ref):
    h1 = jnp.dot(x_ref[...], w1_ref[...], preferred_element_type=f32)
    h1_ref[...] = h1
    acats_ref[...] = jnp.dot(h1, as_ref[...], preferred_element_type=f32)
    acatd_ref[...] = jnp.dot(h1, ad_ref[...], preferred_element_type=f32)


def _dense2_body(op_ref, dp_ref, b1_ref, w2_ref, a2_ref, r16_ref,
                 h2_ref, acat2_ref):
    osum = op_ref[0] + op_ref[1]
    dsum = dp_ref[0] + dp_ref[1]
    den = jnp.dot(dsum, r16_ref[...], preferred_element_type=f32)
    out1 = osum / (den + 1e-16) + b1_ref[...]
    r = jnp.maximum(out1, 0.0)
    h2 = jnp.dot(r, w2_ref[...], preferred_element_type=f32)
    h2_ref[...] = h2
    acat2_ref[...] = jnp.dot(h2, a2_ref[...], preferred_element_type=f32)


def _dense3_body(op_ref, dp_ref, b2_ref, out_ref):
    osum = op_ref[0] + op_ref[1]
    den = dp_ref[0][:, 0:1] + dp_ref[1][:, 0:1]
    logits = osum[:, 0:C_OUT] / (den + 1e-16) + b2_ref[...]
    m = jnp.max(logits, axis=1, keepdims=True)
    s = logits - m
    lse = jnp.log(jnp.sum(jnp.exp(s), axis=1, keepdims=True))
    out_ref[...] = s - lse


def kernel(x, edge_index, W1, att_src1, att_dst1, b1,
           W2, att_src2, att_dst2, b2):
    loop = jnp.arange(N, dtype=jnp.int32)
    pad = jnp.full((EP - E2,), NP - 1, dtype=jnp.int32)
    src = jnp.concatenate([edge_index[0].astype(jnp.int32), loop, pad])
    dst = jnp.concatenate([edge_index[1].astype(jnp.int32), loop, pad])
    xp = jnp.concatenate([x, jnp.zeros((NP - N, F_IN), f32)], axis=0)
    src2 = src.reshape(EP // 128, 128)
    dst2 = dst.reshape(EP // 128, 128)

    eye8 = jnp.eye(H1, dtype=f32)
    a_src_m = jnp.einsum("hc,hk->hck", att_src1, eye8).reshape(D1, H1)
    a_dst_m = jnp.einsum("hc,hk->hck", att_dst1, eye8).reshape(D1, H1)
    z8 = jnp.zeros((D1, 8), f32)
    AS1 = jnp.concatenate([a_src_m, z8], axis=1)
    AD1 = jnp.concatenate([a_dst_m, z8], axis=1)
    R16 = jnp.concatenate(
        [jnp.kron(eye8, jnp.ones((1, C_HID), f32)),
         jnp.zeros((8, D1), f32)], axis=0)
    W2p = jnp.concatenate(
        [W2, jnp.zeros((D1, D2P - C_OUT), f32)], axis=1)
    A2 = jnp.zeros((D2P, 16), f32)
    A2 = A2.at[:C_OUT, 0].set(att_src2[0]).at[:C_OUT, 1].set(att_dst2[0])
    b1r = b1.reshape(1, D1)
    b2r = b2.reshape(1, C_OUT)
    z64 = jnp.zeros((NP, D1), f32)
    z48 = jnp.zeros((NP, D2P), f32)
    z16 = jnp.zeros((NP, 16), f32)

    h1, acats1, acatd1 = pl.pallas_call(
        _dense1_body,
        out_shape=(jax.ShapeDtypeStruct((NP, D1), f32),
                   jax.ShapeDtypeStruct((NP, 16), f32),
                   jax.ShapeDtypeStruct((NP, 16), f32)),
    )(xp, W1, AS1, AD1)

    op1, dp1 = _l1_edge(src2, dst2, acats1, acatd1, h1, z64, z16)

    h2p, acat2 = pl.pallas_call(
        _dense2_body,
        out_shape=(jax.ShapeDtypeStruct((NP, D2P), f32),
                   jax.ShapeDtypeStruct((NP, 16), f32)),
    )(op1, dp1, b1r, W2p, A2, R16)

    asrc2 = acat2[:, 0]
    adst2 = acat2[:, 1]

    op2, dp2 = _l2_edge(src2, dst2, asrc2, adst2, h2p, z48, z16)

    out = pl.pallas_call(
        _dense3_body,
        out_shape=jax.ShapeDtypeStruct((NP, C_OUT), f32),
    )(op2, dp2, b2r)
    return out[:N]


# speedup vs baseline: 71.4223x; 1.1915x over previous
import functools

import jax
import jax.numpy as jnp
from jax import lax
from jax.experimental import pallas as pl
from jax.experimental.pallas import tpu as pltpu
from jax.experimental.pallas import tpu_sc as plsc

N = 10000
F_IN = 128
H1 = 8
C_HID = 8
D1 = H1 * C_HID
C_OUT = 40
D2P = 48
E = 320000
E2 = E + N
NC, NS, LANES = 2, 16, 16
NW = NC * NS
NP = 10240
CHUNK = 256
CQ = CHUNK // 128
NB = 3
K = 42
TE = K * CHUNK
EP = NW * TE
NPT = NP // NS

f32 = jnp.float32

_MESH = plsc.VectorSubcoreMesh(
    core_axis_name="c", subcore_axis_name="s", num_cores=NC, num_subcores=NS
)

_GATHER_DNUMS = lax.GatherDimensionNumbers(
    offset_dims=(), collapsed_slice_dims=(0,), start_index_map=(0,)
)


def _perm16(v, idx):
    return lax.gather(
        v, idx.reshape(16, 1), _GATHER_DNUMS, (1,),
        mode=lax.GatherScatterMode.PROMISE_IN_BOUNDS,
    )


def _l1_body(sd_hbm, acats_hbm, acatd_hbm, h_hbm, z64_hbm, z16_hbm,
             out_hbm, den_hbm,
             idx_v, srow_v, drow_v, hrow_v,
             oacc, dacc, sem_g, sem_w):
    cid = lax.axis_index("c")
    sid = lax.axis_index("s")
    wid = sid * NC + cid
    row0 = sid * NPT

    pltpu.sync_copy(z64_hbm.at[pl.ds(row0, NPT)], oacc.at[pl.ds(row0, NPT)])
    pltpu.sync_copy(z16_hbm.at[pl.ds(row0, NPT)], dacc.at[pl.ds(row0, NPT)])
    plsc.subcore_barrier()

    lane = lax.iota(jnp.int32, 16)
    half = lane >> 3
    scale_idx = [2 * q + half for q in range(4)]

    def gdesc(b):
        cps = []
        for j in range(CQ):
            rows = pl.ds(j * 128, 128)
            sidx = idx_v.at[b, 0, j]
            didx = idx_v.at[b, 1, j]
            cps.append(pltpu.make_async_copy(
                acats_hbm.at[sidx], srow_v.at[b, rows], sem_g.at[b]))
            cps.append(pltpu.make_async_copy(
                acatd_hbm.at[didx], drow_v.at[b, rows], sem_g.at[b]))
            cps.append(pltpu.make_async_copy(
                h_hbm.at[sidx], hrow_v.at[b, rows], sem_g.at[b]))
        return cps

    def wdesc(b):
        cps = []
        for j in range(CQ):
            rows = pl.ds(j * 128, 128)
            didx = idx_v.at[b, 1, j]
            cps.append(pltpu.make_async_copy(
                srow_v.at[b, rows], dacc.at[didx], sem_w.at[b]))
            cps.append(pltpu.make_async_copy(
                hrow_v.at[b, rows], oacc.at[didx], sem_w.at[b]))
        return cps

    pltpu.sync_copy(sd_hbm.at[wid * K], idx_v.at[0])
    for cp in gdesc(0):
        cp.start()

    def outer(kk, carry):
        for b in range(NB):
            nxt = (b + 1) % NB

            def _drain(nb=nxt):
                for cp in wdesc(nb):
                    cp.wait()

            if b < 2:
                pl.when(kk > 0)(_drain)
            else:
                _drain()

            def _pref(nb=nxt, boff=b + 1):
                pltpu.sync_copy(sd_hbm.at[wid * K + kk * NB + boff],
                                idx_v.at[nb])
                for cp in gdesc(nb):
                    cp.start()

            if b == 2:
                pl.when(kk < K // NB - 1)(_pref)
            else:
                _pref()

            for cp in gdesc(b):
                cp.wait()

            @plsc.parallel_loop(0, CHUNK, unroll=8)
            def edge(e, b=b):
                al = srow_v[b, e, :] + drow_v[b, e, :]
                al = jnp.maximum(al, 0.2 * al)
                ex = jnp.exp(al)
                srow_v[b, e, :] = ex
                for q in range(4):
                    scq = _perm16(ex, scale_idx[q])
                    cols = pl.ds(16 * q, 16)
                    hrow_v[b, e, cols] = hrow_v[b, e, cols] * scq

            for cp in wdesc(b):
                cp.start(add=True)
        return carry

    lax.fori_loop(0, K // NB, outer, 0)
    for b in (1, 2):
        for cp in wdesc(b):
            cp.wait()
    plsc.subcore_barrier()
    pltpu.sync_copy(oacc.at[pl.ds(row0, NPT)],
                    out_hbm.at[cid, pl.ds(row0, NPT)])
    pltpu.sync_copy(dacc.at[pl.ds(row0, NPT)],
                    den_hbm.at[cid, pl.ds(row0, NPT)])


_l1_edge = functools.partial(
    pl.kernel,
    out_type=(
        jax.ShapeDtypeStruct((NC, NP, D1), f32),
        jax.ShapeDtypeStruct((NC, NP, 16), f32),
    ),
    mesh=_MESH,
    compiler_params=pltpu.CompilerParams(
        use_tc_tiling_on_sc=False, needs_layout_passes=False),
    scratch_types=[
        pltpu.VMEM((NB, 2, CQ, 128), jnp.int32),
        pltpu.VMEM((NB, CHUNK, 16), f32),
        pltpu.VMEM((NB, CHUNK, 16), f32),
        pltpu.VMEM((NB, CHUNK, D1), f32),
        pltpu.VMEM_SHARED((NP, D1), f32),
        pltpu.VMEM_SHARED((NP, 16), f32),
        pltpu.SemaphoreType.DMA((NB,)),
        pltpu.SemaphoreType.DMA((NB,)),
    ],
)(_l1_body)


def _l2_body(sd_hbm, asrc_hbm, adst_hbm, h_hbm, z48_hbm, z16_hbm,
             out_hbm, den_hbm,
             idx_v, asrc_v, adst_v, hrow_v, exrow_v,
             oacc, dacc, sem_g, sem_w):
    cid = lax.axis_index("c")
    sid = lax.axis_index("s")
    wid = sid * NC + cid
    row0 = sid * NPT

    pltpu.sync_copy(asrc_hbm, asrc_v)
    pltpu.sync_copy(adst_hbm, adst_v)
    pltpu.sync_copy(z48_hbm.at[pl.ds(row0, NPT)], oacc.at[pl.ds(row0, NPT)])
    pltpu.sync_copy(z16_hbm.at[pl.ds(row0, NPT)], dacc.at[pl.ds(row0, NPT)])
    plsc.subcore_barrier()

    def gdesc(b):
        cps = []
        for j in range(CQ):
            rows = pl.ds(j * 128, 128)
            cps.append(pltpu.make_async_copy(
                h_hbm.at[idx_v.at[b, 0, j]], hrow_v.at[b, rows],
                sem_g.at[b]))
        return cps

    def wdesc(b):
        cps = []
        for j in range(CQ):
            rows = pl.ds(j * 128, 128)
            didx = idx_v.at[b, 1, j]
            cps.append(pltpu.make_async_copy(
                exrow_v.at[b, rows], dacc.at[didx], sem_w.at[b]))
            cps.append(pltpu.make_async_copy(
                hrow_v.at[b, rows], oacc.at[didx], sem_w.at[b]))
        return cps

    pltpu.sync_copy(sd_hbm.at[wid * K], idx_v.at[0])
    for cp in gdesc(0):
        cp.start()

    def outer(kk, carry):
        for b in range(NB):
            nxt = (b + 1) % NB

            def _drain(nb=nxt):
                for cp in wdesc(nb):
                    cp.wait()

            if b < 2:
                pl.when(kk > 0)(_drain)
            else:
                _drain()

            def _pref(nb=nxt, boff=b + 1):
                pltpu.sync_copy(sd_hbm.at[wid * K + kk * NB + boff],
                                idx_v.at[nb])
                for cp in gdesc(nb):
                    cp.start()

            if b == 2:
                pl.when(kk < K // NB - 1)(_pref)
            else:
                _pref()

            for cp in gdesc(b):
                cp.wait()

            @plsc.parallel_loop(0, CHUNK // 16, unroll=2)
            def grp(g, b=b):
                src16 = idx_v[b, 0, g // 8, pl.ds((g % 8) * 16, 16)]
                dst16 = idx_v[b, 1, g // 8, pl.ds((g % 8) * 16, 16)]
                a_s = plsc.load_gather(asrc_v, [src16])
                a_d = plsc.load_gather(adst_v, [dst16])
                al = a_s + a_d
                al = jnp.maximum(al, 0.2 * al)
                ex = jnp.exp(al)
                for l in range(16):
                    e = g * 16 + l
                    sc = _perm16(ex, jnp.full((16,), l, jnp.int32))
                    exrow_v[b, e, :] = sc
                    for q in range(3):
                        cols = pl.ds(16 * q, 16)
                        hrow_v[b, e, cols] = hrow_v[b, e, cols] * sc

            for cp in wdesc(b):
                cp.start(add=True)
        return carry

    lax.fori_loop(0, K // NB, outer, 0)
    for b in (1, 2):
        for cp in wdesc(b):
            cp.wait()
    plsc.subcore_barrier()
    pltpu.sync_copy(oacc.at[pl.ds(row0, NPT)],
                    out_hbm.at[cid, pl.ds(row0, NPT)])
    pltpu.sync_copy(dacc.at[pl.ds(row0, NPT)],
                    den_hbm.at[cid, pl.ds(row0, NPT)])


_l2_edge = functools.partial(
    pl.kernel,
    out_type=(
        jax.ShapeDtypeStruct((NC, NP, D2P), f32),
        jax.ShapeDtypeStruct((NC, NP, 16), f32),
    ),
    mesh=_MESH,
    compiler_params=pltpu.CompilerParams(
        use_tc_tiling_on_sc=False, needs_layout_passes=False),
    scratch_types=[
        pltpu.VMEM((NB, 2, CQ, 128), jnp.int32),
        pltpu.VMEM((NP,), f32),
        pltpu.VMEM((NP,), f32),
        pltpu.VMEM((NB, CHUNK, D2P), f32),
        pltpu.VMEM((NB, CHUNK, 16), f32),
        pltpu.VMEM_SHARED((NP, D2P), f32),
        pltpu.VMEM_SHARED((NP, 16), f32),
        pltpu.SemaphoreType.DMA((NB,)),
        pltpu.SemaphoreType.DMA((NB,)),
    ],
)(_l2_body)


def _dense1_body(x_ref, w1_ref, as_ref, ad_ref, h1_ref, acats_ref, acatd_ref):
    h1 = jnp.dot(x_ref[...], w1_ref[...], preferred_element_type=f32)
    h1_ref[...] = h1
    acats_ref[...] = jnp.dot(h1, as_ref[...], preferred_element_type=f32)
    acatd_ref[...] = jnp.dot(h1, ad_ref[...], preferred_element_type=f32)


def _dense2_body(op_ref, dp_ref, b1_ref, w2_ref, a2_ref, r16_ref,
                 h2_ref, acat2_ref):
    osum = op_ref[0] + op_ref[1]
    dsum = dp_ref[0] + dp_ref[1]
    den = jnp.dot(dsum, r16_ref[...], preferred_element_type=f32)
    out1 = osum / (den + 1e-16) + b1_ref[...]
    r = jnp.maximum(out1, 0.0)
    h2 = jnp.dot(r, w2_ref[...], preferred_element_type=f32)
    h2_ref[...] = h2
    acat2_ref[...] = jnp.dot(h2, a2_ref[...], preferred_element_type=f32)


def _dense3_body(op_ref, dp_ref, b2_ref, out_ref):
    osum = op_ref[0] + op_ref[1]
    den = dp_ref[0][:, 0:1] + dp_ref[1][:, 0:1]
    logits = osum[:, 0:C_OUT] / (den + 1e-16) + b2_ref[...]
    m = jnp.max(logits, axis=1, keepdims=True)
    s = logits - m
    lse = jnp.log(jnp.sum(jnp.exp(s), axis=1, keepdims=True))
    out_ref[...] = s - lse


def kernel(x, edge_index, W1, att_src1, att_dst1, b1,
           W2, att_src2, att_dst2, b2):
    loop = jnp.arange(N, dtype=jnp.int32)
    pad = jnp.full((EP - E2,), NP - 1, dtype=jnp.int32)
    src = jnp.concatenate([edge_index[0].astype(jnp.int32), loop, pad])
    dst = jnp.concatenate([edge_index[1].astype(jnp.int32), loop, pad])
    xp = jnp.concatenate([x, jnp.zeros((NP - N, F_IN), f32)], axis=0)
    src3 = src.reshape(EP // CHUNK, CQ, 128)
    dst3 = dst.reshape(EP // CHUNK, CQ, 128)
    sd4 = jnp.stack([src3, dst3], axis=1)

    eye8 = jnp.eye(H1, dtype=f32)
    a_src_m = jnp.einsum("hc,hk->hck", att_src1, eye8).reshape(D1, H1)
    a_dst_m = jnp.einsum("hc,hk->hck", att_dst1, eye8).reshape(D1, H1)
    z8 = jnp.zeros((D1, 8), f32)
    AS1 = jnp.concatenate([a_src_m, z8], axis=1)
    AD1 = jnp.concatenate([a_dst_m, z8], axis=1)
    R16 = jnp.concatenate(
        [jnp.kron(eye8, jnp.ones((1, C_HID), f32)),
         jnp.zeros((8, D1), f32)], axis=0)
    W2p = jnp.concatenate(
        [W2, jnp.zeros((D1, D2P - C_OUT), f32)], axis=1)
    A2 = jnp.zeros((D2P, 16), f32)
    A2 = A2.at[:C_OUT, 0].set(att_src2[0]).at[:C_OUT, 1].set(att_dst2[0])
    b1r = b1.reshape(1, D1)
    b2r = b2.reshape(1, C_OUT)
    z64 = jnp.zeros((NP, D1), f32)
    z48 = jnp.zeros((NP, D2P), f32)
    z16 = jnp.zeros((NP, 16), f32)

    h1, acats1, acatd1 = pl.pallas_call(
        _dense1_body,
        out_shape=(jax.ShapeDtypeStruct((NP, D1), f32),
                   jax.ShapeDtypeStruct((NP, 16), f32),
                   jax.ShapeDtypeStruct((NP, 16), f32)),
    )(xp, W1, AS1, AD1)

    op1, dp1 = _l1_edge(sd4, acats1, acatd1, h1, z64, z16)

    h2p, acat2 = pl.pallas_call(
        _dense2_body,
        out_shape=(jax.ShapeDtypeStruct((NP, D2P), f32),
                   jax.ShapeDtypeStruct((NP, 16), f32)),
    )(op1, dp1, b1r, W2p, A2, R16)

    asrc2 = acat2[:, 0]
    adst2 = acat2[:, 1]

    op2, dp2 = _l2_edge(sd4, asrc2, adst2, h2p, z48, z16)

    out = pl.pallas_call(
        _dense3_body,
        out_shape=jax.ShapeDtypeStruct((NP, C_OUT), f32),
    )(op2, dp2, b2r)
    return out[:N]
